# baseline (speedup 1.0000x reference)
import jax
import jax.numpy as jnp
from jax import lax
from jax.experimental import pallas as pl
from jax.experimental.pallas import tpu as pltpu
from jax.experimental.pallas import tpu_sc as plsc
from jax._src.pallas import mpmd as _mpmd

B, D, M = 16384, 128, 100000
NW = 32
ROWS_A = B // NW
CHUNK = 256
UTILIZATION = float(B % M) / float(M)

_mesh = plsc.VectorSubcoreMesh(core_axis_name="c", subcore_axis_name="s")


def _sc_body(aw_hbm, q_hbm, mem_hbm, util_hbm,
             out_mem, out_util, out_scal,
             bufs, q_v, qfill_v, scal_v,
             in_sem0, in_sem1, out_sem0, out_sem1):
    del mem_hbm, util_hbm
    wid = lax.axis_index("s") * 2 + lax.axis_index("c")
    in_sems = (in_sem0, in_sem1)
    out_sems = (out_sem0, out_sem1)

    a0 = wid * ROWS_A
    n = ROWS_A // CHUNK

    def start_in(i):
        b = i % 2
        return pltpu.async_copy(aw_hbm.at[pl.ds(a0 + i * CHUNK, CHUNK)],
                                bufs.at[b], in_sems[b])

    def start_out(i):
        b = i % 2
        return pltpu.async_copy(bufs.at[b],
                                out_mem.at[pl.ds(a0 + i * CHUNK, CHUNK)],
                                out_sems[b])

    pend_out = [None, None]

    def drain_out(b):
        if pend_out[b] is not None:
            pend_out[b].wait()
            pend_out[b] = None

    h_in = [None, None]
    h_in[0] = start_in(0)
    for i in range(n):
        b = i % 2
        if i + 1 < n:
            nb = (i + 1) % 2
            drain_out(nb)
            h_in[nb] = start_in(i + 1)
        h_in[b].wait()
        pend_out[b] = start_out(i)
    drain_out(0)
    drain_out(1)

    pltpu.sync_copy(q_hbm, q_v.at[pl.ds(0, 1)])
    q = q_v[...][0]
    qvec = jnp.full((16,), q, dtype=jnp.float32)
    for i in range(ROWS_A // 16):
        qfill_v[pl.ds(i * 16, 16)] = qvec
    pltpu.sync_copy(qfill_v, out_util.at[pl.ds(a0, ROWS_A)])

    @pl.when(wid == 0)
    def _write_scalars():
        lane = lax.iota(jnp.int32, 16)
        scal_v[...] = jnp.where(lane == 0, jnp.float32(UTILIZATION), qvec)
        pltpu.sync_copy(scal_v, out_scal)


_sc_update = _mpmd._mpmd_map(
    [(_mesh, _sc_body)],
    out_types=(
        jax.ShapeDtypeStruct((M, D), jnp.float32),
        jax.ShapeDtypeStruct((M,), jnp.float32),
        jax.ShapeDtypeStruct((16,), jnp.float32),
    ),
    input_output_aliases={2: 0, 3: 1},
    scratch_types=[
        pltpu.VMEM((2, CHUNK, D), jnp.float32),
        pltpu.VMEM((16,), jnp.float32),
        pltpu.VMEM((ROWS_A,), jnp.float32),
        pltpu.VMEM((16,), jnp.float32),
        pltpu.SemaphoreType.DMA,
        pltpu.SemaphoreType.DMA,
        pltpu.SemaphoreType.DMA,
        pltpu.SemaphoreType.DMA,
    ],
)


def kernel(features, attention_weights, attention_quality,
           memory_attentions, memory_utilities):
    del features
    new_mem, new_util, scal = _sc_update(
        attention_weights, attention_quality, memory_attentions,
        memory_utilities)
    return (new_mem, new_util, scal[0], scal[1])


# speedup vs baseline: 1.0097x; 1.0097x over previous
import functools

import jax
import jax.numpy as jnp
from jax import lax
from jax.experimental import pallas as pl
from jax.experimental.pallas import tpu as pltpu
from jax.experimental.pallas import tpu_sc as plsc

B, D, M = 16384, 128, 100000
NW = 32
ROWS_A = B // NW
ROWS_B = ((M - B) // NW) // 8 * 8
REM_B = (M - B) - NW * ROWS_B
UTIL_CHUNK = ROWS_B
UTIL_REM = REM_B
CHUNK = 256
NBUF = 3
UTILIZATION = float(B % M) / float(M)

_mesh = plsc.VectorSubcoreMesh(core_axis_name="c", subcore_axis_name="s")


@functools.partial(
    pl.kernel,
    mesh=_mesh,
    out_type=(
        jax.ShapeDtypeStruct((M, D), jnp.float32),
        jax.ShapeDtypeStruct((M,), jnp.float32),
        jax.ShapeDtypeStruct((16,), jnp.float32),
    ),
    scratch_types=[
        pltpu.VMEM((NBUF, CHUNK, D), jnp.float32),
        pltpu.VMEM((16,), jnp.float32),
        pltpu.VMEM((ROWS_A,), jnp.float32),
        pltpu.VMEM((16,), jnp.float32),
        pltpu.VMEM((UTIL_CHUNK,), jnp.float32),
        pltpu.VMEM((REM_B, D), jnp.float32),
        pltpu.VMEM((UTIL_REM,), jnp.float32),
        pltpu.SemaphoreType.DMA,
        pltpu.SemaphoreType.DMA,
        pltpu.SemaphoreType.DMA,
        pltpu.SemaphoreType.DMA,
        pltpu.SemaphoreType.DMA,
        pltpu.SemaphoreType.DMA,
        pltpu.SemaphoreType.DMA,
        pltpu.SemaphoreType.DMA,
        pltpu.SemaphoreType.DMA,
    ],
)
def _sc_update(aw_hbm, q_hbm, mem_hbm, util_hbm,
               out_mem, out_util, out_scal,
               bufs, q_v, qfill_v, scal_v, util_v, mrem_v, urem_v,
               in_sem0, in_sem1, in_sem2, out_sem0, out_sem1, out_sem2,
               q_sem, uin_sem, uout_sem):
    wid = lax.axis_index("s") * 2 + lax.axis_index("c")
    in_sems = (in_sem0, in_sem1, in_sem2)
    out_sems = (out_sem0, out_sem1, out_sem2)

    h_q = pltpu.async_copy(q_hbm, q_v.at[pl.ds(0, 1)], q_sem)
    u0 = B + wid * UTIL_CHUNK
    h_uin = pltpu.async_copy(util_hbm.at[pl.ds(u0, UTIL_CHUNK)], util_v,
                             uin_sem)

    items = [(aw_hbm, wid * ROWS_A, CHUNK),
             (aw_hbm, wid * ROWS_A + CHUNK, CHUNK)]
    tail0 = B + wid * ROWS_B
    nfull, last = divmod(ROWS_B, CHUNK)
    for j in range(nfull):
        items.append((mem_hbm, tail0 + j * CHUNK, CHUNK))
    if last:
        items.append((mem_hbm, tail0 + nfull * CHUNK, last))
    n = len(items)

    def start_in(i):
        src, off, rows = items[i]
        b = i % NBUF
        return pltpu.async_copy(src.at[pl.ds(off, rows)],
                                bufs.at[b, pl.ds(0, rows)], in_sems[b])

    def start_out(i):
        _, off, rows = items[i]
        b = i % NBUF
        return pltpu.async_copy(bufs.at[b, pl.ds(0, rows)],
                                out_mem.at[pl.ds(off, rows)], out_sems[b])

    pend_out = [None] * NBUF
    pend_in = [None] * NBUF

    def drain_out(b):
        if pend_out[b] is not None:
            pend_out[b].wait()
            pend_out[b] = None

    for i in range(NBUF - 1):
        pend_in[i % NBUF] = start_in(i)
    for i in range(n):
        b = i % NBUF
        if i + NBUF - 1 < n:
            nb = (i + NBUF - 1) % NBUF
            drain_out(nb)
            pend_in[nb] = start_in(i + NBUF - 1)
        pend_in[b].wait()
        pend_out[b] = start_out(i)
    for b in range(NBUF):
        drain_out(b)

    @pl.when(wid == NW - 2)
    def _copy_mem_remainder():
        r0 = B + NW * ROWS_B
        pltpu.sync_copy(mem_hbm.at[pl.ds(r0, REM_B)], mrem_v)
        pltpu.sync_copy(mrem_v, out_mem.at[pl.ds(r0, REM_B)])

    h_q.wait()
    q = q_v[...][0]
    qvec = jnp.full((16,), q, dtype=jnp.float32)
    for i in range(ROWS_A // 16):
        qfill_v[pl.ds(i * 16, 16)] = qvec
    h_qout = pltpu.async_copy(qfill_v, out_util.at[pl.ds(wid * ROWS_A, ROWS_A)],
                              uout_sem)

    h_uin.wait()
    h_uout = pltpu.async_copy(util_v, out_util.at[pl.ds(u0, UTIL_CHUNK)],
                              uout_sem)

    @pl.when(wid == NW - 1)
    def _copy_util_remainder():
        r0 = B + NW * UTIL_CHUNK
        pltpu.sync_copy(util_hbm.at[pl.ds(r0, UTIL_REM)], urem_v)
        pltpu.sync_copy(urem_v, out_util.at[pl.ds(r0, UTIL_REM)])

    @pl.when(wid == 0)
    def _write_scalars():
        lane = lax.iota(jnp.int32, 16)
        scal_v[...] = jnp.where(lane == 0, jnp.float32(UTILIZATION), qvec)
        pltpu.sync_copy(scal_v, out_scal)

    h_qout.wait()
    h_uout.wait()


def kernel(features, attention_weights, attention_quality,
           memory_attentions, memory_utilities):
    del features
    new_mem, new_util, scal = _sc_update(
        attention_weights, attention_quality, memory_attentions,
        memory_utilities)
    return (new_mem, new_util, scal[0], scal[1])
